# baseline (speedup 1.0000x reference)
import functools

import jax
import jax.numpy as jnp
from jax.experimental import pallas as pl
from jax.experimental.pallas import tpu as pltpu

E = 64
D = 2048
TILE = 256
NSTEPS = D // TILE


def _router_kernel(idx_ref,
                   x_ref,
                   gamma_ref, beta_ref,
                   w1_ref,
                   b1_ref,
                   w2_ref,
                   b2_ref,
                   emb_ref,
                   cent_ref,
                   spat_ref,
                   comp_ref,
                   out_ref,
                   xln_ref,
                   acc_ref):
    j = pl.program_id(0)

    @pl.when(j == 0)
    def _init():
        x = x_ref[:, 7, :]
        mu = jnp.mean(x, axis=-1, keepdims=True)
        var = jnp.mean((x - mu) ** 2, axis=-1, keepdims=True)
        xln_ref[...] = ((x - mu) * jax.lax.rsqrt(var + 1e-5)
                        * gamma_ref[...] + beta_ref[...])
        acc_ref[...] = jnp.zeros_like(acc_ref)

    xln = xln_ref[...]
    h1 = jnp.dot(xln, w1_ref[...], preferred_element_type=jnp.float32,
                 precision=jax.lax.Precision.DEFAULT) + b1_ref[...]
    h1 = 0.5 * h1 * (1.0 + jax.lax.erf(h1 * 0.7071067811865476))
    acc_ref[...] += jnp.dot(h1, w2_ref[...], preferred_element_type=jnp.float32,
                            precision=jax.lax.Precision.DEFAULT)

    @pl.when(j == NSTEPS - 1)
    def _finish():
        h2 = acc_ref[...] + b2_ref[...]
        att = jax.lax.dot_general(
            h2, emb_ref[...], (((1,), (1,)), ((), ())),
            preferred_element_type=jnp.float32,
            precision=jax.lax.Precision.DEFAULT) * (1.0 / (D ** 0.5))
        cent = cent_ref[...]
        cent = jax.nn.softmax(cent, axis=-1)
        idx = idx_ref[0, 0]
        row = spat_ref[pl.ds(idx, 1), :] + comp_ref[pl.ds(idx, 1), :]
        eids = jax.lax.broadcasted_iota(jnp.int32, (1, E), 1)
        row = row + jnp.where(eids == idx, -0.1, 0.0)
        att = att + cent + row
        out_ref[...] = jax.nn.softmax(att, axis=-1)


def kernel(hidden_states, ln_gamma, ln_beta, W1, b1, W2, b2,
           expert_embeddings, centrality_bias, spatial_bias,
           compatibility_matrix, current_expert_idx):
    B, S, d = hidden_states.shape
    idx = jnp.asarray(current_expert_idx, jnp.int32).reshape(1, 1)
    grid_spec = pltpu.PrefetchScalarGridSpec(
        num_scalar_prefetch=1,
        grid=(NSTEPS,),
        in_specs=[
            pl.BlockSpec((B, 8, d), lambda j, *_: (0, S // 8 - 1, 0)),
            pl.BlockSpec((1, d), lambda j, *_: (0, 0)),
            pl.BlockSpec((1, d), lambda j, *_: (0, 0)),
            pl.BlockSpec((d, TILE), lambda j, *_: (0, j)),
            pl.BlockSpec((1, TILE), lambda j, *_: (0, j)),
            pl.BlockSpec((TILE, d), lambda j, *_: (j, 0)),
            pl.BlockSpec((1, d), lambda j, *_: (0, 0)),
            pl.BlockSpec((E, d), lambda j, *_: (0, 0)),
            pl.BlockSpec((1, E), lambda j, *_: (0, 0)),
            pl.BlockSpec((E, E), lambda j, *_: (0, 0)),
            pl.BlockSpec((E, E), lambda j, *_: (0, 0)),
        ],
        out_specs=pl.BlockSpec((B, E), lambda j, *_: (0, 0)),
        scratch_shapes=[
            pltpu.VMEM((B, d), jnp.float32),
            pltpu.VMEM((B, d), jnp.float32),
        ],
    )
    return pl.pallas_call(
        _router_kernel,
        grid_spec=grid_spec,
        out_shape=jax.ShapeDtypeStruct((B, E), jnp.float32),
        compiler_params=pltpu.CompilerParams(
            dimension_semantics=("arbitrary",),
        ),
    )(idx,
      hidden_states,
      ln_gamma.reshape(1, d), ln_beta.reshape(1, d),
      W1, b1.reshape(1, d),
      W2, b2.reshape(1, d),
      expert_embeddings,
      centrality_bias.reshape(1, E),
      spatial_bias, compatibility_matrix)


# speedup vs baseline: 1.1419x; 1.1419x over previous
import functools

import jax
import jax.numpy as jnp
from jax.experimental import pallas as pl
from jax.experimental.pallas import tpu as pltpu

E = 64
D = 2048
TILE = 512
NSTEPS = D // TILE


def _router_kernel(idx_ref,
                   x_ref,
                   gamma_ref, beta_ref,
                   w1_ref,
                   b1_ref,
                   w2_ref,
                   b2_ref,
                   emb_ref,
                   cent_ref,
                   spat_ref,
                   comp_ref,
                   out_ref,
                   xln_ref,
                   acc_ref):
    j = pl.program_id(0)

    @pl.when(j == 0)
    def _init():
        x = x_ref[:, 7, :]
        mu = jnp.mean(x, axis=-1, keepdims=True)
        var = jnp.mean((x - mu) ** 2, axis=-1, keepdims=True)
        xln_ref[...] = ((x - mu) * jax.lax.rsqrt(var + 1e-5)
                        * gamma_ref[...] + beta_ref[...])
        acc_ref[...] = jnp.zeros_like(acc_ref)

    xln = xln_ref[...]
    h1 = jnp.dot(xln, w1_ref[...], preferred_element_type=jnp.float32,
                 precision=jax.lax.Precision.DEFAULT) + b1_ref[...]
    h1 = 0.5 * h1 * (1.0 + jax.lax.erf(h1 * 0.7071067811865476))
    acc_ref[...] += jnp.dot(h1, w2_ref[...], preferred_element_type=jnp.float32,
                            precision=jax.lax.Precision.DEFAULT)

    @pl.when(j == NSTEPS - 1)
    def _finish():
        h2 = acc_ref[...] + b2_ref[...]
        att = jax.lax.dot_general(
            h2, emb_ref[...], (((1,), (1,)), ((), ())),
            preferred_element_type=jnp.float32,
            precision=jax.lax.Precision.DEFAULT) * (1.0 / (D ** 0.5))
        cent = cent_ref[...]
        cent = jax.nn.softmax(cent, axis=-1)
        idx = idx_ref[0, 0]
        row = spat_ref[pl.ds(idx, 1), :] + comp_ref[pl.ds(idx, 1), :]
        eids = jax.lax.broadcasted_iota(jnp.int32, (1, E), 1)
        row = row + jnp.where(eids == idx, -0.1, 0.0)
        att = att + cent + row
        out_ref[...] = jax.nn.softmax(att, axis=-1)


def kernel(hidden_states, ln_gamma, ln_beta, W1, b1, W2, b2,
           expert_embeddings, centrality_bias, spatial_bias,
           compatibility_matrix, current_expert_idx):
    B, S, d = hidden_states.shape
    idx = jnp.asarray(current_expert_idx, jnp.int32).reshape(1, 1)
    grid_spec = pltpu.PrefetchScalarGridSpec(
        num_scalar_prefetch=1,
        grid=(NSTEPS,),
        in_specs=[
            pl.BlockSpec((B, 8, d), lambda j, *_: (0, S // 8 - 1, 0)),
            pl.BlockSpec((1, d), lambda j, *_: (0, 0)),
            pl.BlockSpec((1, d), lambda j, *_: (0, 0)),
            pl.BlockSpec((d, TILE), lambda j, *_: (0, j)),
            pl.BlockSpec((1, TILE), lambda j, *_: (0, j)),
            pl.BlockSpec((TILE, d), lambda j, *_: (j, 0)),
            pl.BlockSpec((1, d), lambda j, *_: (0, 0)),
            pl.BlockSpec((E, d), lambda j, *_: (0, 0)),
            pl.BlockSpec((1, E), lambda j, *_: (0, 0)),
            pl.BlockSpec((E, E), lambda j, *_: (0, 0)),
            pl.BlockSpec((E, E), lambda j, *_: (0, 0)),
        ],
        out_specs=pl.BlockSpec((B, E), lambda j, *_: (0, 0)),
        scratch_shapes=[
            pltpu.VMEM((B, d), jnp.float32),
            pltpu.VMEM((B, d), jnp.float32),
        ],
    )
    return pl.pallas_call(
        _router_kernel,
        grid_spec=grid_spec,
        out_shape=jax.ShapeDtypeStruct((B, E), jnp.float32),
        compiler_params=pltpu.CompilerParams(
            dimension_semantics=("arbitrary",),
        ),
    )(idx,
      hidden_states,
      ln_gamma.reshape(1, d), ln_beta.reshape(1, d),
      W1, b1.reshape(1, d),
      W2, b2.reshape(1, d),
      expert_embeddings,
      centrality_bias.reshape(1, E),
      spatial_bias, compatibility_matrix)
